# Initial kernel scaffold; baseline (speedup 1.0000x reference)
import jax
import jax.numpy as jnp
from jax.experimental import pallas as pl


def kernel(interp_points, values_points, values):
    raise NotImplementedError("write your pallas kernel here")


# speedup vs baseline: 5.8584x; 5.8584x over previous
import functools

import jax
import jax.numpy as jnp
from jax import lax
from jax.experimental import pallas as pl
from jax.experimental.pallas import tpu as pltpu
from jax.experimental.pallas import tpu_sc as plsc

M = 4096
N = 16384
B = 64

MB = 256
NC = 2048

SC_CORES = 2
SC_SUBCORES = 16
NW = SC_CORES * SC_SUBCORES
BPW = M // NW


def _argmin_body(aT_ref, b_ref, idx_ref, min_ref, arg_ref):
    j = pl.program_id(1)
    a0 = aT_ref[:, 0:1]
    a1 = aT_ref[:, 1:2]
    diag1 = a0 * a0 + a1 * a1
    a0b = a0.astype(jnp.bfloat16).astype(jnp.float32)
    a1b = a1.astype(jnp.bfloat16).astype(jnp.float32)
    b0 = b_ref[0:1, :]
    b1 = b_ref[1:2, :]
    diag2 = b0 * b0 + b1 * b1
    b0b = b0.astype(jnp.bfloat16).astype(jnp.float32)
    b1b = b1.astype(jnp.bfloat16).astype(jnp.float32)
    r = a0b * b0b + a1b * b1b
    s = (diag1 + diag2) - 2.0 * r
    s = jnp.where(s >= 0.0, s, jnp.inf)
    cmin = jnp.min(s, axis=1, keepdims=True)
    iota = lax.broadcasted_iota(jnp.int32, (MB, NC), 1) + j * NC
    cidx = jnp.min(
        jnp.where(s == cmin, iota, jnp.int32(2**30)),
        axis=1, keepdims=True)

    @pl.when(j == 0)
    def _():
        min_ref[:, :] = cmin
        arg_ref[:, :] = cidx

    @pl.when(j > 0)
    def _():
        better = cmin < min_ref[:, :]
        min_ref[:, :] = jnp.where(better, cmin, min_ref[:, :])
        arg_ref[:, :] = jnp.where(better, cidx, arg_ref[:, :])

    @pl.when(j == pl.num_programs(1) - 1)
    def _():
        idx_ref[:, :] = arg_ref[:, :]


def _nearest_idx(aT, b):
    return pl.pallas_call(
        _argmin_body,
        grid=(M // MB, N // NC),
        in_specs=[
            pl.BlockSpec((MB, 2), lambda i, j: (i, 0)),
            pl.BlockSpec((2, NC), lambda i, j: (0, j)),
        ],
        out_specs=pl.BlockSpec((MB, 1), lambda i, j: (i, 0)),
        out_shape=jax.ShapeDtypeStruct((M, 1), jnp.int32),
        scratch_shapes=[
            pltpu.VMEM((MB, 1), jnp.float32),
            pltpu.VMEM((MB, 1), jnp.int32),
        ],
    )(aT, b)


def _gather_body(table_hbm, idx_hbm, out_hbm, idx_v, rows_v, sem):
    wid = lax.axis_index("s") * SC_CORES + lax.axis_index("c")
    base = wid * BPW
    pltpu.sync_copy(idx_hbm.at[pl.ds(base, BPW)], idx_v)
    pltpu.async_copy(table_hbm.at[idx_v], rows_v, sem).wait()
    pltpu.sync_copy(rows_v, out_hbm.at[pl.ds(base, BPW)])


BP = 128


@functools.cache
def _sc_gather():
    return pl.kernel(
        _gather_body,
        out_type=jax.ShapeDtypeStruct((M, BP), jnp.float32),
        mesh=plsc.VectorSubcoreMesh(
            core_axis_name="c", subcore_axis_name="s",
            num_cores=SC_CORES, num_subcores=SC_SUBCORES),
        scratch_types=[
            pltpu.VMEM((BPW,), jnp.int32),
            pltpu.VMEM((BPW, BP), jnp.float32),
            pltpu.SemaphoreType.DMA,
        ],
    )


def kernel(interp_points, values_points, values):
    aT = interp_points.T
    idx = _nearest_idx(aT, values_points)
    tableT = jnp.pad(values, ((0, BP - B), (0, 0))).T
    rows = _sc_gather()(tableT, idx.reshape(M))
    return rows[:, :B].T[:, :, None]
